# Initial kernel scaffold; baseline (speedup 1.0000x reference)
import jax
import jax.numpy as jnp
from jax.experimental import pallas as pl


def kernel(features, indices, W1c, gamma_c, beta_c, mean_c, var_c, W2c, b2c, W1o, gamma_o, beta_o, mean_o, var_o, W2o, b2o):
    raise NotImplementedError("write your pallas kernel here")


# speedup vs baseline: 1.6994x; 1.6994x over previous
import jax
import jax.numpy as jnp
from jax import lax
from jax.experimental import pallas as pl
from jax.experimental.pallas import tpu as pltpu

_D = 128
_NCLS = 3
_K = 2048
_NB = 4
_EPS = 1e-5
_BN = 1024


def _mlp_block(feat_ref, idx_ref, w12t_ref, gam_ref, bet_ref, mu_ref, var_ref,
               w2ct_ref, b2c_ref, w2ot_ref, b2o_ref, masked_ref, payload_ref):
    x = feat_ref[...]
    h = jnp.dot(x, w12t_ref[...], preferred_element_type=jnp.float32)
    h = gam_ref[...] * (h - mu_ref[...]) / jnp.sqrt(var_ref[...] + _EPS) + bet_ref[...]
    h = jnp.maximum(h, 0.0)
    hc = h[:, :_D]
    ho = h[:, _D:]
    s8 = jnp.dot(hc, w2ct_ref[...], preferred_element_type=jnp.float32) + b2c_ref[...]
    o8 = jnp.dot(ho, w2ot_ref[...], preferred_element_type=jnp.float32) + b2o_ref[...]
    off = o8[:, :2] * 16.0 / 8.0
    lim = jnp.clip(jnp.ceil(off), -3.0, 3.0)
    bidx = idx_ref[:, 0:1]
    bf = bidx.astype(jnp.float32)
    votes12 = idx_ref[:, 1:3].astype(jnp.float32) + lim
    p3 = jax.nn.sigmoid(s8[:, :3])
    p12 = jnp.concatenate([p3, p3, p3, p3], axis=1)
    r = lax.broadcasted_iota(jnp.int32, (1, 12), 1) // 3
    masked12 = jnp.where(bidx == r, p12, -jnp.inf)
    neg4 = jnp.full((x.shape[0], 4), -jnp.inf, jnp.float32)
    masked_ref[...] = jnp.concatenate([masked12, neg4], axis=1)
    zeros10 = jnp.zeros((x.shape[0], 10), jnp.float32)
    payload_ref[...] = jnp.concatenate([s8[:, :3], bf, votes12, zeros10], axis=1)


def kernel(features, indices, W1c, gamma_c, beta_c, mean_c, var_c, W2c, b2c,
           W1o, gamma_o, beta_o, mean_o, var_o, W2o, b2o):
    n = features.shape[0]
    nb = n // _BN
    idxp = jnp.pad(indices, ((0, 0), (0, 1)))
    w12t = jnp.concatenate([W1c.T, W1o.T], axis=1)
    gam = jnp.concatenate([gamma_c, gamma_o]).reshape(1, 2 * _D)
    bet = jnp.concatenate([beta_c, beta_o]).reshape(1, 2 * _D)
    mu = jnp.concatenate([mean_c, mean_o]).reshape(1, 2 * _D)
    var = jnp.concatenate([var_c, var_o]).reshape(1, 2 * _D)
    w2ct = jnp.zeros((_D, 8), jnp.float32).at[:, :3].set(W2c.T)
    b2cp = jnp.zeros((1, 8), jnp.float32).at[0, :3].set(b2c)
    w2ot = jnp.zeros((_D, 8), jnp.float32).at[:, :2].set(W2o.T)
    b2op = jnp.zeros((1, 8), jnp.float32).at[0, :2].set(b2o)

    rep = lambda shape: pl.BlockSpec(shape, lambda i: (0, 0))
    masked, payload = pl.pallas_call(
        _mlp_block,
        grid=(nb,),
        in_specs=[
            pl.BlockSpec((_BN, _D), lambda i: (i, 0)),
            pl.BlockSpec((_BN, 4), lambda i: (i, 0)),
            rep((_D, 2 * _D)), rep((1, 2 * _D)), rep((1, 2 * _D)),
            rep((1, 2 * _D)), rep((1, 2 * _D)),
            rep((_D, 8)), rep((1, 8)), rep((_D, 8)), rep((1, 8)),
        ],
        out_specs=[pl.BlockSpec((_BN, 16), lambda i: (i, 0)),
                   pl.BlockSpec((_BN, 16), lambda i: (i, 0))],
        out_shape=[jax.ShapeDtypeStruct((n, 16), jnp.float32),
                   jax.ShapeDtypeStruct((n, 16), jnp.float32)],
        compiler_params=pltpu.CompilerParams(
            dimension_semantics=("arbitrary",)),
    )(features, idxp, w12t, gam, bet, mu, var, w2ct, b2cp, w2ot, b2op)

    maskedT = masked.T[:_NB * _NCLS]
    tk = lax.top_k(maskedT, _K)[1]
    tkflat = tk.reshape(_NB, _NCLS, _K).transpose(0, 2, 1).reshape(-1)
    gp = payload[tkflat]
    gf = features[tkflat]
    votes = gp[:, 3:6].reshape(_NB, _K, _NCLS, 3)
    scores = gp[:, 0:3].reshape(_NB, _K, _NCLS, 3)
    feats = gf.reshape(_NB, _K, _NCLS, _D)
    return votes, feats, scores
